# Initial kernel scaffold; baseline (speedup 1.0000x reference)
import jax
import jax.numpy as jnp
from jax.experimental import pallas as pl


def kernel(hidden_states, token_ids, Wq, Wk, Wv, Wo, q_norm_w, k_norm_w, ln1_w, ln2_w, gate_proj, up_proj, down_proj):
    raise NotImplementedError("write your pallas kernel here")


# speedup vs baseline: 1.9220x; 1.9220x over previous
import functools
import math

import jax
import jax.numpy as jnp
from jax import lax
from jax.experimental import pallas as pl
from jax.experimental.pallas import tpu as pltpu
from jax.experimental.pallas import tpu_sc as plsc

S = 2048
HID = 768
NH = 12
NKV = 4
DH = 64
HALF = DH // 2
GROUPS = NH // NKV
NEXP = 64
EI = 48
VOCAB = 100000
THETA = 10000.0
EPS = 1e-6

SBLK = 256
NSB = S // SBLK
BT = 128
NBLOCKS = NEXP + S // BT
NPAD = NBLOCKS * BT

NW = 32
ROWS_W = S // NW


def _rms(x, w):
    return x * lax.rsqrt(jnp.mean(x * x, axis=-1, keepdims=True) + EPS) * w


def _qkv_body(hid_ref, wq_ref, wk_ref, wv_ref, ln1_ref, qnw_ref, knw_ref,
              q_ref, k_ref, v_ref):
    i = pl.program_id(0)
    x = hid_ref[...]
    h = _rms(x, ln1_ref[...])
    q = jnp.dot(h, wq_ref[...], preferred_element_type=jnp.float32)
    k = jnp.dot(h, wk_ref[...], preferred_element_type=jnp.float32)
    v = jnp.dot(h, wv_ref[...], preferred_element_type=jnp.float32)
    t = (lax.broadcasted_iota(jnp.int32, (SBLK, HALF), 0) + i * SBLK
         ).astype(jnp.float32)
    j = lax.broadcasted_iota(jnp.int32, (SBLK, HALF), 1).astype(jnp.float32)
    freqs = t * jnp.exp(j * (-math.log(THETA) / HALF))
    cos = jnp.cos(freqs)
    sin = jnp.sin(freqs)

    def rope_norm(xc, w):
        x1 = xc[:, :HALF]
        x2 = xc[:, HALF:]
        r = jnp.concatenate([x1 * cos - x2 * sin, x2 * cos + x1 * sin], axis=1)
        return _rms(r, w)

    for h_ in range(NH):
        q_ref[h_] = rope_norm(q[:, h_ * DH:(h_ + 1) * DH], qnw_ref[...])
    for h_ in range(NKV):
        k_ref[h_] = rope_norm(k[:, h_ * DH:(h_ + 1) * DH], knw_ref[...])
        v_ref[h_] = v[:, h_ * DH:(h_ + 1) * DH]


def _attn_body(q_ref, k_ref, v_ref, o_ref):
    i = pl.program_id(1)
    q = q_ref[0]
    k = k_ref[0]
    s = lax.dot_general(q, k, (((1,), (1,)), ((), ())),
                        preferred_element_type=jnp.float32) * (1.0 / math.sqrt(DH))
    row = lax.broadcasted_iota(jnp.int32, (SBLK, S), 0) + i * SBLK
    col = lax.broadcasted_iota(jnp.int32, (SBLK, S), 1)
    s = jnp.where(col <= row, s, jnp.float32(-1e30))
    m = jnp.max(s, axis=-1, keepdims=True)
    p = jnp.exp(s - m)
    p = p / jnp.sum(p, axis=-1, keepdims=True)
    o_ref[0] = jnp.dot(p, v_ref[0], preferred_element_type=jnp.float32)


def _oproj_body(a_ref, wo_ref, r_ref, x_ref):
    a = jnp.concatenate([a_ref[h_] for h_ in range(NH)], axis=1)
    x_ref[...] = r_ref[...] + jnp.dot(a, wo_ref[...],
                                      preferred_element_type=jnp.float32)


def _moe_body(be_ref, x_ref, ln2_ref, g_ref, u_ref, d_ref, y_ref):
    del be_ref
    x = x_ref[...]
    h = _rms(x, ln2_ref[...])
    g = jnp.dot(h, g_ref[0], preferred_element_type=jnp.float32)
    u = jnp.dot(h, u_ref[0], preferred_element_type=jnp.float32)
    a = g * (1.0 / (1.0 + jnp.exp(-g))) * u
    y_ref[...] = x + jnp.dot(a, d_ref[0], preferred_element_type=jnp.float32)


def _sc_mesh():
    return plsc.VectorSubcoreMesh(core_axis_name="c", subcore_axis_name="s")


def _sc_scatter_rows(x2d, dest_idx):
    @functools.partial(
        pl.kernel, mesh=_sc_mesh(),
        out_type=jax.ShapeDtypeStruct((NPAD, HID), jnp.float32),
        scratch_types=[pltpu.VMEM((ROWS_W,), jnp.int32),
                       pltpu.VMEM((ROWS_W, HID), jnp.float32),
                       pltpu.SemaphoreType.DMA],
    )
    def k(x_hbm, idx_hbm, out_hbm, idx_v, rows_v, sem):
        wid = lax.axis_index("s") * 2 + lax.axis_index("c")
        base = wid * ROWS_W
        pltpu.sync_copy(idx_hbm.at[pl.ds(base, ROWS_W)], idx_v)
        pltpu.sync_copy(x_hbm.at[pl.ds(base, ROWS_W)], rows_v)
        pltpu.async_copy(rows_v, out_hbm.at[idx_v], sem).wait()

    return k(x2d, dest_idx)


def _sc_gather_rows(y_padded, dest_idx):
    @functools.partial(
        pl.kernel, mesh=_sc_mesh(),
        out_type=jax.ShapeDtypeStruct((S, HID), jnp.float32),
        scratch_types=[pltpu.VMEM((ROWS_W,), jnp.int32),
                       pltpu.VMEM((ROWS_W, HID), jnp.float32),
                       pltpu.SemaphoreType.DMA],
    )
    def k(y_hbm, idx_hbm, out_hbm, idx_v, rows_v, sem):
        wid = lax.axis_index("s") * 2 + lax.axis_index("c")
        base = wid * ROWS_W
        pltpu.sync_copy(idx_hbm.at[pl.ds(base, ROWS_W)], idx_v)
        pltpu.async_copy(y_hbm.at[idx_v], rows_v, sem).wait()
        pltpu.sync_copy(rows_v, out_hbm.at[pl.ds(base, ROWS_W)])

    return k(y_padded, dest_idx)


def kernel(hidden_states, token_ids, Wq, Wk, Wv, Wo, q_norm_w, k_norm_w,
           ln1_w, ln2_w, gate_proj, up_proj, down_proj):
    x0 = hidden_states.reshape(S, HID)

    tid = jnp.clip(token_ids.reshape(-1), 0, VOCAB - 1)
    eid = jnp.minimum(tid // (VOCAB // NEXP), NEXP - 1).astype(jnp.int32)
    order = jnp.argsort(eid)
    eid_sorted = eid[order]
    counts = jnp.bincount(eid, length=NEXP).astype(jnp.int32)
    blocks_per_e = (counts + BT - 1) // BT
    cumblocks = jnp.cumsum(blocks_per_e)
    pstart = (cumblocks - blocks_per_e) * BT
    gstart = jnp.cumsum(counts) - counts
    rank = jnp.arange(S, dtype=jnp.int32) - gstart[eid_sorted]
    dest_sorted = pstart[eid_sorted] + rank
    dest_idx = jnp.zeros((S,), jnp.int32).at[order].set(dest_sorted.astype(jnp.int32))
    block_expert = jnp.minimum(
        jnp.searchsorted(cumblocks, jnp.arange(NBLOCKS, dtype=jnp.int32),
                         side='right'),
        NEXP - 1).astype(jnp.int32)

    qn2d, kn2d, v2d = pl.pallas_call(
        _qkv_body,
        grid=(NSB,),
        in_specs=[
            pl.BlockSpec((SBLK, HID), lambda i: (i, 0)),
            pl.BlockSpec((HID, NH * DH), lambda i: (0, 0)),
            pl.BlockSpec((HID, NKV * DH), lambda i: (0, 0)),
            pl.BlockSpec((HID, NKV * DH), lambda i: (0, 0)),
            pl.BlockSpec((1, HID), lambda i: (0, 0)),
            pl.BlockSpec((1, DH), lambda i: (0, 0)),
            pl.BlockSpec((1, DH), lambda i: (0, 0)),
        ],
        out_specs=[
            pl.BlockSpec((NH, SBLK, DH), lambda i: (0, i, 0)),
            pl.BlockSpec((NKV, SBLK, DH), lambda i: (0, i, 0)),
            pl.BlockSpec((NKV, SBLK, DH), lambda i: (0, i, 0)),
        ],
        out_shape=[
            jax.ShapeDtypeStruct((NH, S, DH), jnp.float32),
            jax.ShapeDtypeStruct((NKV, S, DH), jnp.float32),
            jax.ShapeDtypeStruct((NKV, S, DH), jnp.float32),
        ],
    )(x0, Wq, Wk, Wv, ln1_w.reshape(1, HID), q_norm_w.reshape(1, DH),
      k_norm_w.reshape(1, DH))

    attn3 = pl.pallas_call(
        _attn_body,
        grid=(NH, NSB),
        in_specs=[
            pl.BlockSpec((1, SBLK, DH), lambda h, i: (h, i, 0)),
            pl.BlockSpec((1, S, DH), lambda h, i: (h // GROUPS, 0, 0)),
            pl.BlockSpec((1, S, DH), lambda h, i: (h // GROUPS, 0, 0)),
        ],
        out_specs=pl.BlockSpec((1, SBLK, DH), lambda h, i: (h, i, 0)),
        out_shape=jax.ShapeDtypeStruct((NH, S, DH), jnp.float32),
    )(qn2d, kn2d, v2d)

    x2d = pl.pallas_call(
        _oproj_body,
        grid=(NSB,),
        in_specs=[
            pl.BlockSpec((NH, SBLK, DH), lambda i: (0, i, 0)),
            pl.BlockSpec((NH * DH, HID), lambda i: (0, 0)),
            pl.BlockSpec((SBLK, HID), lambda i: (i, 0)),
        ],
        out_specs=pl.BlockSpec((SBLK, HID), lambda i: (i, 0)),
        out_shape=jax.ShapeDtypeStruct((S, HID), jnp.float32),
    )(attn3, Wo, x0)

    x_padded = _sc_scatter_rows(x2d, dest_idx)

    y_padded = pl.pallas_call(
        _moe_body,
        grid_spec=pltpu.PrefetchScalarGridSpec(
            num_scalar_prefetch=1,
            grid=(NBLOCKS,),
            in_specs=[
                pl.BlockSpec((BT, HID), lambda b, be: (b, 0)),
                pl.BlockSpec((1, HID), lambda b, be: (0, 0)),
                pl.BlockSpec((1, HID, EI), lambda b, be: (be[b], 0, 0)),
                pl.BlockSpec((1, HID, EI), lambda b, be: (be[b], 0, 0)),
                pl.BlockSpec((1, EI, HID), lambda b, be: (be[b], 0, 0)),
            ],
            out_specs=pl.BlockSpec((BT, HID), lambda b, be: (b, 0)),
        ),
        out_shape=jax.ShapeDtypeStruct((NPAD, HID), jnp.float32),
    )(block_expert, x_padded, ln2_w.reshape(1, HID), gate_proj, up_proj,
      down_proj)

    out2d = _sc_gather_rows(y_padded, dest_idx)
    return out2d.reshape(1, S, HID)
